# baseline (speedup 1.0000x reference)
import functools

import jax
import jax.numpy as jnp
from jax import lax
from jax.experimental import pallas as pl
from jax.experimental.pallas import tpu as pltpu
from jax.experimental.pallas import tpu_sc as plsc

B = 16384
DIM = 128
N_DOY = 366
N_HOUR = 24
NC = 2
NS = 16
NW = NC * NS

B_SC = 8192
B_TC = B - B_SC
B_PER_W = B_SC // NW
CHUNK = 128
N_CHUNKS = B_PER_W // CHUNK

TC_BLK = 1024
N_TC_BLKS = B_TC // TC_BLK


def _tc_build(day_ref, hour_ref, doy_t_ref, hour_t_ref, w_ref, b_ref,
              cross_ref, idx_ref, doyp_ref, hourpb_ref):
    w = w_ref[...]
    doy_proj = lax.dot_general(
        doy_t_ref[...], w[:, :DIM],
        (((1,), (1,)), ((), ())), preferred_element_type=jnp.float32)
    hour_proj = lax.dot_general(
        hour_t_ref[...], w[:, DIM:],
        (((1,), (1,)), ((), ())), preferred_element_type=jnp.float32)
    hourpb = hour_proj + b_ref[...]
    doyp_ref[...] = doy_proj
    hourpb_ref[...] = hourpb
    cross_ref[...] = doy_proj[:, None, :] + hourpb[None, :, :]
    d = jnp.clip(day_ref[...], 0, N_DOY - 1)
    h = jnp.clip(hour_ref[...], 0, N_HOUR - 1)
    idx_ref[...] = d * N_HOUR + h


def _tc_onehot(day_ref, hour_ref, doyp_ref, hourpb_ref, out_ref):
    d = jnp.clip(day_ref[...], 0, N_DOY - 1)
    h = jnp.clip(hour_ref[...], 0, N_HOUR - 1)
    oh_d = (d == lax.broadcasted_iota(jnp.int32, (TC_BLK, N_DOY), 1)
            ).astype(jnp.float32)
    oh_h = (h == lax.broadcasted_iota(jnp.int32, (TC_BLK, N_HOUR), 1)
            ).astype(jnp.float32)
    out_ref[...] = (
        lax.dot_general(oh_d, doyp_ref[...],
                        (((1,), (0,)), ((), ())),
                        preferred_element_type=jnp.float32)
        + lax.dot_general(oh_h, hourpb_ref[...],
                          (((1,), (0,)), ((), ())),
                          preferred_element_type=jnp.float32))


@functools.cache
def _make_sc_gather():
    mesh = plsc.VectorSubcoreMesh(core_axis_name="c", subcore_axis_name="s")

    @functools.partial(
        pl.kernel,
        mesh=mesh,
        out_type=jax.ShapeDtypeStruct((B_SC, DIM), jnp.float32),
        scratch_types=[
            pltpu.VMEM((N_CHUNKS, CHUNK), jnp.int32),
            pltpu.VMEM((B_PER_W, DIM), jnp.float32),
            pltpu.SemaphoreType.DMA,
            pltpu.SemaphoreType.DMA,
        ],
    )
    def _sc_gather(table_hbm, idx_hbm, out_hbm, idx_v, rows_v, gsem, wsem):
        wid = lax.axis_index("s") * NC + lax.axis_index("c")
        base = wid * B_PER_W
        pltpu.sync_copy(idx_hbm.at[wid], idx_v)
        gathers = [
            pltpu.async_copy(table_hbm.at[idx_v.at[j]],
                             rows_v.at[pl.ds(j * CHUNK, CHUNK)], gsem)
            for j in range(N_CHUNKS)
        ]
        writes = []
        for j in range(N_CHUNKS):
            gathers[j].wait()
            writes.append(
                pltpu.async_copy(rows_v.at[pl.ds(j * CHUNK, CHUNK)],
                                 out_hbm.at[pl.ds(base + j * CHUNK, CHUNK)],
                                 wsem))
        for w in writes:
            w.wait()

    return _sc_gather


def kernel(day_of_year, hour_of_day, doy_table, hour_table, W, b):
    day = day_of_year.astype(jnp.int32)
    hour = hour_of_day.astype(jnp.int32)
    day_sc = day[:B_SC].reshape(64, 128)
    hour_sc = hour[:B_SC].reshape(64, 128)
    cross, idx, doyp, hourpb = pl.pallas_call(
        _tc_build,
        out_shape=(
            jax.ShapeDtypeStruct((N_DOY, N_HOUR, DIM), jnp.float32),
            jax.ShapeDtypeStruct((64, 128), jnp.int32),
            jax.ShapeDtypeStruct((N_DOY, DIM), jnp.float32),
            jax.ShapeDtypeStruct((N_HOUR, DIM), jnp.float32),
        ),
    )(day_sc, hour_sc, doy_table, hour_table, W, b.reshape(1, DIM))
    out_sc = _make_sc_gather()(cross.reshape(N_DOY * N_HOUR, DIM),
                               idx.reshape(NW, N_CHUNKS, CHUNK))
    out_tc = pl.pallas_call(
        _tc_onehot,
        grid=(N_TC_BLKS,),
        in_specs=[
            pl.BlockSpec((TC_BLK, 1), lambda i: (i, 0)),
            pl.BlockSpec((TC_BLK, 1), lambda i: (i, 0)),
            pl.BlockSpec((N_DOY, DIM), lambda i: (0, 0)),
            pl.BlockSpec((N_HOUR, DIM), lambda i: (0, 0)),
        ],
        out_specs=pl.BlockSpec((TC_BLK, DIM), lambda i: (i, 0)),
        out_shape=jax.ShapeDtypeStruct((B_TC, DIM), jnp.float32),
    )(day[B_SC:].reshape(B_TC, 1), hour[B_SC:].reshape(B_TC, 1),
      doyp, hourpb)
    return jnp.concatenate([out_sc, out_tc], axis=0)


# speedup vs baseline: 1.6006x; 1.6006x over previous
import functools

import jax
import jax.numpy as jnp
from jax import lax
from jax.experimental import pallas as pl
from jax.experimental.pallas import tpu as pltpu
from jax.experimental.pallas import tpu_sc as plsc

B = 16384
DIM = 128
N_DOY = 366
N_HOUR = 24
NC = 2
NS = 16
NW = NC * NS

B_PER_W = B // NW
CHUNK = 128
N_CHUNKS = B_PER_W // CHUNK


def _tc_build(day_ref, hour_ref, doy_t_ref, hour_t_ref, w_ref, b_ref,
              cross_ref, idx_ref):
    w = w_ref[...]
    doy_proj = lax.dot_general(
        doy_t_ref[...], w[:, :DIM],
        (((1,), (1,)), ((), ())), preferred_element_type=jnp.float32)
    hour_proj = lax.dot_general(
        hour_t_ref[...], w[:, DIM:],
        (((1,), (1,)), ((), ())), preferred_element_type=jnp.float32)
    cross_ref[...] = doy_proj[:, None, :] + (hour_proj + b_ref[...])[None, :, :]
    d = jnp.clip(day_ref[...], 0, N_DOY - 1)
    h = jnp.clip(hour_ref[...], 0, N_HOUR - 1)
    idx_ref[...] = d * N_HOUR + h


@functools.cache
def _make_sc_gather():
    mesh = plsc.VectorSubcoreMesh(core_axis_name="c", subcore_axis_name="s")

    @functools.partial(
        pl.kernel,
        mesh=mesh,
        out_type=jax.ShapeDtypeStruct((B, DIM), jnp.float32),
        scratch_types=[
            pltpu.VMEM((N_CHUNKS, CHUNK), jnp.int32),
            pltpu.VMEM((B_PER_W, DIM), jnp.float32),
            pltpu.SemaphoreType.DMA,
            pltpu.SemaphoreType.DMA,
        ],
    )
    def _sc_gather(table_hbm, idx_hbm, out_hbm, idx_v, rows_v, gsem, wsem):
        wid = lax.axis_index("s") * NC + lax.axis_index("c")
        base = wid * B_PER_W
        pltpu.sync_copy(idx_hbm.at[wid], idx_v)
        gathers = [
            pltpu.async_copy(table_hbm.at[idx_v.at[j]],
                             rows_v.at[pl.ds(j * CHUNK, CHUNK)], gsem)
            for j in range(N_CHUNKS)
        ]
        writes = []
        for j in range(N_CHUNKS):
            gathers[j].wait()
            writes.append(
                pltpu.async_copy(rows_v.at[pl.ds(j * CHUNK, CHUNK)],
                                 out_hbm.at[pl.ds(base + j * CHUNK, CHUNK)],
                                 wsem))
        for w in writes:
            w.wait()

    return _sc_gather


def kernel(day_of_year, hour_of_day, doy_table, hour_table, W, b):
    day = day_of_year.astype(jnp.int32).reshape(B // CHUNK, CHUNK)
    hour = hour_of_day.astype(jnp.int32).reshape(B // CHUNK, CHUNK)
    cross, idx = pl.pallas_call(
        _tc_build,
        out_shape=(
            jax.ShapeDtypeStruct((N_DOY, N_HOUR, DIM), jnp.float32),
            jax.ShapeDtypeStruct((B // CHUNK, CHUNK), jnp.int32),
        ),
    )(day, hour, doy_table, hour_table, W, b.reshape(1, DIM))
    return _make_sc_gather()(cross.reshape(N_DOY * N_HOUR, DIM),
                             idx.reshape(NW, N_CHUNKS, CHUNK))
